# baseline (speedup 1.0000x reference)
import jax
import jax.numpy as jnp
import numpy as np
from jax.experimental import pallas as pl
from jax.experimental.pallas import tpu as pltpu
from jax.experimental.shard_map import shard_map
from jax.sharding import Mesh, NamedSharding, PartitionSpec as P

S, E, C = 4096, 8, 1024
TILE_S = 512

_devs = jax.devices()
_NDEV = 2 if (len(_devs) >= 2 and _devs[0].platform == "tpu") else 1
_S_LOCAL = S // _NDEV


def _fused_kernel(logits_ref, m1_ref, m2_ref, loc1_ref, loc2_ref,
                  acc_out_ref, combine_ref, acc_ref):
    i = pl.program_id(0)
    n = pl.num_programs(0)

    lg = logits_ref[...]
    m1 = m1_ref[...]
    m2 = m2_ref[...]

    mx = jnp.max(lg, axis=1, keepdims=True)
    ex = jnp.exp(lg - mx)
    gates = ex / jnp.sum(ex, axis=1, keepdims=True)

    @pl.when(i == 0)
    def _():
        acc_ref[...] = jnp.zeros_like(acc_ref)

    acc_ref[0:1, :] += jnp.sum(gates, axis=0, keepdims=True)
    acc_ref[1:2, :] += jnp.sum(m1, axis=0, keepdims=True)

    g1s = jnp.sum(gates * m1, axis=1, keepdims=True)
    g2s = jnp.sum(gates * m2, axis=1, keepdims=True)
    denom = jnp.maximum(g1s + g2s, jnp.finfo(jnp.float32).eps)
    g1 = (g1s / denom) * m1
    g2 = (g2s / denom) * m2

    loc1 = loc1_ref[...]
    loc2 = loc2_ref[...]
    out = g1[:, :, None] * loc1[:, None, :] + g2[:, :, None] * loc2[:, None, :]
    combine_ref[...] = out

    @pl.when(i == n - 1)
    def _():
        acc_out_ref[...] = acc_ref[...]


def _run_local(logits, m1, m2, loc1, loc2, s_local):
    grid = (s_local // TILE_S,)
    return pl.pallas_call(
        _fused_kernel,
        grid=grid,
        in_specs=[
            pl.BlockSpec((TILE_S, E), lambda i: (i, 0)),
            pl.BlockSpec((TILE_S, E), lambda i: (i, 0)),
            pl.BlockSpec((TILE_S, E), lambda i: (i, 0)),
            pl.BlockSpec((TILE_S, C), lambda i: (i, 0)),
            pl.BlockSpec((TILE_S, C), lambda i: (i, 0)),
        ],
        out_specs=[
            pl.BlockSpec((2, E), lambda i: (0, 0)),
            pl.BlockSpec((TILE_S, E, C), lambda i: (i, 0, 0)),
        ],
        out_shape=[
            jax.ShapeDtypeStruct((2, E), jnp.float32),
            jax.ShapeDtypeStruct((s_local, E, C), jnp.float32),
        ],
        scratch_shapes=[pltpu.VMEM((2, E), jnp.float32)],
        compiler_params=pltpu.CompilerParams(
            dimension_semantics=("arbitrary",),
        ),
    )(logits, m1, m2, loc1, loc2)


def _laux_from_acc(acc):
    scale = jnp.float32(E) / jnp.float32(S * S)
    return jnp.sum(acc[0, :] * acc[1, :]) * scale


if _NDEV >= 2:
    _mesh = Mesh(np.array(_devs[:_NDEV]), ("x",))

    def _sharded(logits, m1, m2, loc1, loc2):
        acc, combine = _run_local(logits, m1, m2, loc1, loc2, _S_LOCAL)
        acc = jax.lax.psum(acc, "x")
        return _laux_from_acc(acc), combine

    _sharded_fn = shard_map(
        _sharded,
        mesh=_mesh,
        in_specs=(P("x"), P("x"), P("x"), P("x"), P("x")),
        out_specs=(P(), P("x")),
        check_rep=False,
    )

    def kernel(logits, mask1_float, mask2_float, locations1_sc, locations2_sc):
        sh = NamedSharding(_mesh, P("x"))
        args = [jax.lax.with_sharding_constraint(a, sh)
                for a in (logits, mask1_float, mask2_float,
                          locations1_sc, locations2_sc)]
        return _sharded_fn(*args)
else:
    def kernel(logits, mask1_float, mask2_float, locations1_sc, locations2_sc):
        acc, combine = _run_local(logits, mask1_float, mask2_float,
                                  locations1_sc, locations2_sc, S)
        return _laux_from_acc(acc), combine


# speedup vs baseline: 10.1455x; 10.1455x over previous
import functools

import jax
import jax.numpy as jnp
from jax.experimental import pallas as pl
from jax.experimental.pallas import tpu as pltpu

S, E, C = 4096, 8, 1024
TILE_S = 512
CHUNK = 16


def _fused_kernel(logits_ref, m1_ref, m2_ref, loc1_ref, loc2_ref,
                  laux_ref, combine_ref, acc_ref):
    i = pl.program_id(0)
    n = pl.num_programs(0)

    lg = logits_ref[...]
    m1 = m1_ref[...]
    m2 = m2_ref[...]

    mx = jnp.max(lg, axis=1, keepdims=True)
    ex = jnp.exp(lg - mx)
    gates = ex / jnp.sum(ex, axis=1, keepdims=True)

    @pl.when(i == 0)
    def _():
        acc_ref[...] = jnp.zeros_like(acc_ref)

    acc_ref[0:1, :] += jnp.sum(gates, axis=0, keepdims=True)
    acc_ref[1:2, :] += jnp.sum(m1, axis=0, keepdims=True)

    g1s = jnp.sum(gates * m1, axis=1, keepdims=True)
    g2s = jnp.sum(gates * m2, axis=1, keepdims=True)
    denom = jnp.maximum(g1s + g2s, jnp.finfo(jnp.float32).eps)
    g1 = (g1s / denom) * m1
    g2 = (g2s / denom) * m2

    for j in range(TILE_S // CHUNK):
        sl = slice(j * CHUNK, (j + 1) * CHUNK)
        g1c = g1[sl, :, None]
        g2c = g2[sl, :, None]
        l1c = loc1_ref[sl, :][:, None, :]
        l2c = loc2_ref[sl, :][:, None, :]
        combine_ref[sl, :, :] = g1c * l1c + g2c * l2c

    @pl.when(i == n - 1)
    def _():
        me_ce = acc_ref[0:1, :] * acc_ref[1:2, :]
        scale = jnp.float32(E) / jnp.float32(S * S)
        laux_ref[...] = jnp.sum(me_ce, axis=1, keepdims=True) * scale


@functools.partial(jax.jit, static_argnames=("interpret",))
def kernel(logits, mask1_float, mask2_float, locations1_sc, locations2_sc,
           interpret=False):
    grid = (S // TILE_S,)
    laux, combine = pl.pallas_call(
        _fused_kernel,
        grid=grid,
        in_specs=[
            pl.BlockSpec((TILE_S, E), lambda i: (i, 0)),
            pl.BlockSpec((TILE_S, E), lambda i: (i, 0)),
            pl.BlockSpec((TILE_S, E), lambda i: (i, 0)),
            pl.BlockSpec((TILE_S, C), lambda i: (i, 0)),
            pl.BlockSpec((TILE_S, C), lambda i: (i, 0)),
        ],
        out_specs=[
            pl.BlockSpec((1, 1), lambda i: (0, 0)),
            pl.BlockSpec((TILE_S, E, C), lambda i: (i, 0, 0)),
        ],
        out_shape=[
            jax.ShapeDtypeStruct((1, 1), jnp.float32),
            jax.ShapeDtypeStruct((S, E, C), jnp.float32),
        ],
        scratch_shapes=[pltpu.VMEM((2, E), jnp.float32)],
        compiler_params=pltpu.CompilerParams(
            dimension_semantics=("arbitrary",),
        ),
        interpret=interpret,
    )(logits, mask1_float, mask2_float, locations1_sc, locations2_sc)
    return laux[0, 0], combine


# speedup vs baseline: 10.1608x; 1.0015x over previous
import functools

import jax
import jax.numpy as jnp
from jax.experimental import pallas as pl
from jax.experimental.pallas import tpu as pltpu

S, E, C = 4096, 8, 1024
TILE_S = 512
CHUNK = 2


def _fused_kernel(logits_ref, m1_ref, m2_ref, loc1_ref, loc2_ref,
                  laux_ref, combine_ref, acc_ref):
    i = pl.program_id(0)
    n = pl.num_programs(0)

    lg = logits_ref[...]
    m1 = m1_ref[...]
    m2 = m2_ref[...]

    mx = jnp.max(lg, axis=1, keepdims=True)
    ex = jnp.exp(lg - mx)
    gates = ex / jnp.sum(ex, axis=1, keepdims=True)

    @pl.when(i == 0)
    def _():
        acc_ref[...] = jnp.zeros_like(acc_ref)

    acc_ref[0:1, :] += jnp.sum(gates, axis=0, keepdims=True)
    acc_ref[1:2, :] += jnp.sum(m1, axis=0, keepdims=True)

    g1s = jnp.sum(gates * m1, axis=1, keepdims=True)
    g2s = jnp.sum(gates * m2, axis=1, keepdims=True)
    denom = jnp.maximum(g1s + g2s, jnp.finfo(jnp.float32).eps)
    g1 = (g1s / denom) * m1
    g2 = (g2s / denom) * m2

    for j in range(TILE_S // CHUNK):
        sl = slice(j * CHUNK, (j + 1) * CHUNK)
        g1c = g1[sl, :, None]
        g2c = g2[sl, :, None]
        l1c = loc1_ref[sl, :][:, None, :]
        l2c = loc2_ref[sl, :][:, None, :]
        combine_ref[sl, :, :] = g1c * l1c + g2c * l2c

    @pl.when(i == n - 1)
    def _():
        me_ce = acc_ref[0:1, :] * acc_ref[1:2, :]
        scale = jnp.float32(E) / jnp.float32(S * S)
        laux_ref[...] = jnp.sum(me_ce, axis=1, keepdims=True) * scale


@functools.partial(jax.jit, static_argnames=("interpret",))
def kernel(logits, mask1_float, mask2_float, locations1_sc, locations2_sc,
           interpret=False):
    grid = (S // TILE_S,)
    laux, combine = pl.pallas_call(
        _fused_kernel,
        grid=grid,
        in_specs=[
            pl.BlockSpec((TILE_S, E), lambda i: (i, 0)),
            pl.BlockSpec((TILE_S, E), lambda i: (i, 0)),
            pl.BlockSpec((TILE_S, E), lambda i: (i, 0)),
            pl.BlockSpec((TILE_S, C), lambda i: (i, 0)),
            pl.BlockSpec((TILE_S, C), lambda i: (i, 0)),
        ],
        out_specs=[
            pl.BlockSpec((1, 1), lambda i: (0, 0)),
            pl.BlockSpec((TILE_S, E, C), lambda i: (i, 0, 0)),
        ],
        out_shape=[
            jax.ShapeDtypeStruct((1, 1), jnp.float32),
            jax.ShapeDtypeStruct((S, E, C), jnp.float32),
        ],
        scratch_shapes=[pltpu.VMEM((2, E), jnp.float32)],
        compiler_params=pltpu.CompilerParams(
            dimension_semantics=("arbitrary",),
        ),
        interpret=interpret,
    )(logits, mask1_float, mask2_float, locations1_sc, locations2_sc)
    return laux[0, 0], combine
